# baseline (speedup 1.0000x reference)
import functools

import jax
import jax.numpy as jnp
from jax import lax
from jax.experimental import pallas as pl
from jax.experimental.pallas import tpu as pltpu
from jax.experimental.pallas import tpu_sc as plsc

_NC = 2
_NS = 16
_NW = _NC * _NS


@functools.partial(jax.jit, static_argnames=("chunk",))
def _gather(idx_flat, lookup_table, chunk):
    B = idx_flat.shape[0]
    D = lookup_table.shape[1]
    b_per_w = B // _NW
    n_chunks = b_per_w // chunk
    mesh = plsc.VectorSubcoreMesh(
        core_axis_name="c", subcore_axis_name="s",
        num_cores=_NC, num_subcores=_NS,
    )

    @functools.partial(
        pl.kernel,
        out_type=jax.ShapeDtypeStruct((B, D), jnp.float32),
        mesh=mesh,
        scratch_types=[
            pltpu.VMEM((chunk,), jnp.int32),
            pltpu.VMEM((chunk, D), jnp.float32),
            pltpu.SemaphoreType.DMA,
        ],
        compiler_params=pltpu.CompilerParams(use_tc_tiling_on_sc=False),
    )
    def k(idx_hbm, table_hbm, out_hbm, idx_v, rows_v, sem):
        wid = lax.axis_index("s") * _NC + lax.axis_index("c")
        base = wid * b_per_w

        def body(i, carry):
            off = base + i * chunk
            pltpu.sync_copy(idx_hbm.at[pl.ds(off, chunk)], idx_v)
            pltpu.async_copy(table_hbm.at[idx_v], rows_v, sem).wait()
            pltpu.sync_copy(rows_v, out_hbm.at[pl.ds(off, chunk)])
            return carry

        lax.fori_loop(0, n_chunks, body, 0)

    return k(idx_flat, lookup_table)


def kernel(indices, lookup_table):
    B0, H = indices.shape
    D = lookup_table.shape[1]
    idx_flat = indices.reshape(B0 * H).astype(jnp.int32)
    out = _gather(idx_flat, lookup_table, chunk=800)
    return out.reshape(B0, H, D)


# speedup vs baseline: 1.0061x; 1.0061x over previous
import functools

import jax
import jax.numpy as jnp
from jax import lax
from jax.experimental import pallas as pl
from jax.experimental.pallas import tpu as pltpu
from jax.experimental.pallas import tpu_sc as plsc

_NC = 2
_NS = 16
_NW = _NC * _NS


@functools.partial(jax.jit, static_argnames=("chunk",))
def _gather(idx_flat, lookup_table, chunk):
    B = idx_flat.shape[0]
    D = lookup_table.shape[1]
    b_per_w = B // _NW
    n_chunks = b_per_w // chunk
    mesh = plsc.VectorSubcoreMesh(
        core_axis_name="c", subcore_axis_name="s",
        num_cores=_NC, num_subcores=_NS,
    )

    @functools.partial(
        pl.kernel,
        out_type=jax.ShapeDtypeStruct((B, D), jnp.float32),
        mesh=mesh,
        scratch_types=[
            pltpu.VMEM((b_per_w,), jnp.int32),
            pltpu.VMEM((2, chunk, D), jnp.float32),
            pltpu.SemaphoreType.DMA,
            pltpu.SemaphoreType.DMA,
            pltpu.SemaphoreType.DMA,
            pltpu.SemaphoreType.DMA,
        ],
        compiler_params=pltpu.CompilerParams(use_tc_tiling_on_sc=False),
    )
    def k(idx_hbm, table_hbm, out_hbm, idx_v, rows_v, g0, g1, w0, w1):
        wid = lax.axis_index("s") * _NC + lax.axis_index("c")
        base = wid * b_per_w
        gsem = (g0, g1)
        wsem = (w0, w1)

        pltpu.sync_copy(idx_hbm.at[pl.ds(base, b_per_w)], idx_v)

        def start_gather(i):
            b = i % 2
            pltpu.async_copy(
                table_hbm.at[idx_v.at[pl.ds(i * chunk, chunk)]],
                rows_v.at[b], gsem[b])

        start_gather(0)
        for i in range(n_chunks):
            b = i % 2
            pltpu.make_async_copy(
                table_hbm.at[idx_v.at[pl.ds(i * chunk, chunk)]],
                rows_v.at[b], gsem[b]).wait()
            if i + 1 < n_chunks:
                if i >= 1:
                    pltpu.make_async_copy(
                        rows_v.at[1 - b],
                        out_hbm.at[pl.ds(base + (i - 1) * chunk, chunk)],
                        wsem[1 - b]).wait()
                start_gather(i + 1)
            pltpu.async_copy(
                rows_v.at[b],
                out_hbm.at[pl.ds(base + i * chunk, chunk)], wsem[b])
        for i in (n_chunks - 2, n_chunks - 1):
            b = i % 2
            pltpu.make_async_copy(
                rows_v.at[b],
                out_hbm.at[pl.ds(base + i * chunk, chunk)], wsem[b]).wait()

    return k(idx_flat, lookup_table)


def kernel(indices, lookup_table):
    B0, H = indices.shape
    D = lookup_table.shape[1]
    idx_flat = indices.reshape(B0 * H).astype(jnp.int32)
    out = _gather(idx_flat, lookup_table, chunk=800)
    return out.reshape(B0, H, D)


# speedup vs baseline: 2.2638x; 2.2501x over previous
import functools

import jax
import jax.numpy as jnp
from jax import lax
from jax.experimental import pallas as pl
from jax.experimental.pallas import tpu as pltpu
from jax.experimental.pallas import tpu_sc as plsc

_NC = 2
_NS = 16


@jax.jit
def _gather(idx_t, table_t):
    H, B = idx_t.shape
    D, V = table_t.shape
    d_per_core = D // _NC
    b_chunk = B // _NS
    n_idx = H * b_chunk
    v_main = (V // _NS) // 128 * 128
    v_tail = V - v_main * _NS

    mesh = plsc.VectorSubcoreMesh(
        core_axis_name="c", subcore_axis_name="s",
        num_cores=_NC, num_subcores=_NS,
    )

    @functools.partial(
        pl.kernel,
        out_type=jax.ShapeDtypeStruct((H, D, B), jnp.float32),
        mesh=mesh,
        scratch_types=[
            pltpu.VMEM((n_idx,), jnp.int32),
            pltpu.VMEM((n_idx,), jnp.float32),
            pltpu.VMEM((n_idx,), jnp.float32),
            pltpu.VMEM_SHARED((1, V), jnp.float32),
            pltpu.SemaphoreType.DMA,
            pltpu.SemaphoreType.DMA,
            pltpu.SemaphoreType.DMA,
            pltpu.SemaphoreType.DMA,
        ],
    )
    def k(idx_hbm, table_hbm, out_hbm, idx_v, val_a, val_b, row_sh,
          ssem, gsem, wsem_a, wsem_b):
        c = lax.axis_index("c")
        s = lax.axis_index("s")
        b0 = pl.multiple_of(s * b_chunk, 128)
        v0 = pl.multiple_of(s * v_main, 128)
        d_base = c * d_per_core

        for h in range(H):
            pltpu.sync_copy(idx_hbm.at[h, pl.ds(b0, b_chunk)],
                            idx_v.at[pl.ds(h * b_chunk, b_chunk)])

        def stage_descr(row):
            return pltpu.make_async_copy(
                table_hbm.at[pl.ds(row, 1), pl.ds(v0, v_main)],
                row_sh.at[:, pl.ds(v0, v_main)], ssem)

        def tail_descr(row):
            return pltpu.make_async_copy(
                table_hbm.at[pl.ds(row, 1), pl.ds(v_main * _NS, v_tail)],
                row_sh.at[:, pl.ds(v_main * _NS, v_tail)], ssem)

        def process(d, val_v, wsem):
            row = d_base + d
            stage_descr(row).start()
            @pl.when(s == _NS - 1)
            def _():
                tail_descr(row).start()
            @pl.when(d >= 2)
            def _():
                pltpu.make_async_copy(
                    table_hbm.at[0, pl.ds(0, n_idx)], val_v, wsem).wait()
            stage_descr(row).wait()
            @pl.when(s == _NS - 1)
            def _():
                tail_descr(row).wait()
            plsc.subcore_barrier()
            pltpu.make_async_copy(row_sh.at[0].at[idx_v], val_v, gsem).start()
            pltpu.make_async_copy(row_sh.at[0].at[idx_v], val_v, gsem).wait()
            plsc.subcore_barrier()
            for h in range(H):
                pltpu.make_async_copy(
                    val_v.at[pl.ds(h * b_chunk, b_chunk)],
                    out_hbm.at[h, row, pl.ds(b0, b_chunk)], wsem).start()

        def body(i, carry):
            d = i * 2
            process(d, val_a, wsem_a)
            process(d + 1, val_b, wsem_b)
            return carry

        lax.fori_loop(0, d_per_core // 2, body, 0)
        pltpu.make_async_copy(
            table_hbm.at[0, pl.ds(0, n_idx)], val_a, wsem_a).wait()
        pltpu.make_async_copy(
            table_hbm.at[0, pl.ds(0, n_idx)], val_b, wsem_b).wait()

    return k(idx_t, table_t)


def kernel(indices, lookup_table):
    out3 = _gather(indices.T, lookup_table.T)
    return out3.transpose(2, 0, 1)


# speedup vs baseline: 2.9336x; 1.2959x over previous
import functools

import jax
import jax.numpy as jnp
from jax import lax
from jax.experimental import pallas as pl
from jax.experimental.pallas import tpu as pltpu
from jax.experimental.pallas import tpu_sc as plsc

_NC = 2
_NS = 16
_KH = 5


@functools.partial(jax.jit, static_argnames=("H", "B"))
def _gather(idx_tiles, table_t, H, B):
    D, V = table_t.shape
    d_per_core = D // _NC
    b_chunk = B // _NS
    nq = H // _KH
    cs = _KH * b_chunk
    v_main = (V // _NS) // 128 * 128
    v_tail = V - v_main * _NS

    mesh = plsc.VectorSubcoreMesh(
        core_axis_name="c", subcore_axis_name="s",
        num_cores=_NC, num_subcores=_NS,
    )

    @functools.partial(
        pl.kernel,
        out_type=jax.ShapeDtypeStruct((H, D, B), jnp.float32),
        mesh=mesh,
        scratch_types=[
            pltpu.VMEM((cs,), jnp.int32),
            pltpu.VMEM((cs,), jnp.int32),
            pltpu.VMEM((cs,), jnp.float32),
            pltpu.VMEM((cs,), jnp.float32),
            pltpu.VMEM_SHARED((1, V), jnp.float32),
            pltpu.VMEM_SHARED((1, V), jnp.float32),
            pltpu.SemaphoreType.DMA,
            pltpu.SemaphoreType.DMA,
            pltpu.SemaphoreType.DMA,
            pltpu.SemaphoreType.DMA,
            pltpu.SemaphoreType.DMA,
            pltpu.SemaphoreType.DMA,
        ],
    )
    def k(idx_hbm, table_hbm, out_hbm, idx0, idx1, val0, val1, row_a, row_b,
          ssem_a, ssem_b, isem, gsem, wsem0, wsem1):
        c = lax.axis_index("c")
        s = lax.axis_index("s")
        b0 = pl.multiple_of(s * b_chunk, 128)
        v0 = pl.multiple_of(s * v_main, 128)
        d_base = c * d_per_core
        idxb = (idx0, idx1)
        valb = (val0, val1)
        wsem = (wsem0, wsem1)

        def idx_copy(q, buf):
            return pltpu.make_async_copy(
                idx_hbm.at[s, pl.ds((q % nq) * cs, cs)], buf, isem)

        def stage_descrs(row, row_sh, ssem):
            return (
                pltpu.make_async_copy(
                    table_hbm.at[pl.ds(row, 1), pl.ds(v0, v_main)],
                    row_sh.at[:, pl.ds(v0, v_main)], ssem),
                pltpu.make_async_copy(
                    table_hbm.at[pl.ds(row, 1), pl.ds(v_main * _NS, v_tail)],
                    row_sh.at[:, pl.ds(v_main * _NS, v_tail)], ssem),
            )

        def fire_stage(row, row_sh, ssem):
            ds_ = stage_descrs(row, row_sh, ssem)
            ds_[0].start()
            @pl.when(s == _NS - 1)
            def _():
                ds_[1].start()

        def wait_stage(row, row_sh, ssem):
            ds_ = stage_descrs(row, row_sh, ssem)
            ds_[0].wait()
            @pl.when(s == _NS - 1)
            def _():
                ds_[1].wait()

        def drain(sem, buf):
            pltpu.make_async_copy(table_hbm.at[0, pl.ds(0, cs)], buf,
                                  sem).wait()

        def fire_writes(q, row):
            p = q % 2
            for i in range(_KH):
                h = q * _KH + i
                pltpu.make_async_copy(
                    valb[p].at[pl.ds(i * b_chunk, b_chunk)],
                    out_hbm.at[h, row, pl.ds(b0, b_chunk)], wsem[p]).start()

        def process(d, row_sh, other_sh, ssem, other_ssem):
            row = d_base + d
            plsc.subcore_barrier()
            @pl.when(d + 1 < d_per_core)
            def _():
                fire_stage(row + 1, other_sh, other_ssem)
            wait_stage(row, row_sh, ssem)
            plsc.subcore_barrier()
            for q in range(nq):
                p = q % 2
                if q >= 2:
                    drain(wsem[p], valb[p])
                else:
                    @pl.when(d > 0)
                    def _():
                        drain(wsem[p], valb[p])
                if q > 0:
                    drain(gsem, valb[1 - p])
                idx_copy(q, idxb[p]).wait()
                idx_copy(q + 1, idxb[(q + 1) % 2]).start()
                pltpu.make_async_copy(row_sh.at[0].at[idxb[p]], valb[p],
                                      gsem).start()
                if q > 0:
                    fire_writes(q - 1, row)
            drain(gsem, valb[(nq - 1) % 2])
            fire_writes(nq - 1, row)

        idx_copy(0, idxb[0]).start()
        fire_stage(d_base, row_a, ssem_a)

        def body(i, carry):
            d = i * 2
            process(d, row_a, row_b, ssem_a, ssem_b)
            process(d + 1, row_b, row_a, ssem_b, ssem_a)
            return carry

        lax.fori_loop(0, d_per_core // 2, body, 0)
        idx_copy(0, idxb[0]).wait()
        drain(wsem[0], valb[0])
        drain(wsem[1], valb[1])

    return k(idx_tiles, table_t)


def kernel(indices, lookup_table):
    B0, H = indices.shape
    idx_tiles = (indices.T.reshape(H, _NS, B0 // _NS)
                 .transpose(1, 0, 2).reshape(_NS, H * (B0 // _NS)))
    out3 = _gather(idx_tiles, lookup_table.T, H, B0)
    return out3.transpose(2, 0, 1)
